# baseline (speedup 1.0000x reference)
import functools

import jax
import jax.numpy as jnp
from jax import lax
from jax.experimental import pallas as pl
from jax.experimental.pallas import tpu as pltpu
from jax.experimental.pallas import tpu_sc as plsc

N_NODES = 10000
N_EDGES = 160000
IN_CH = 128
HID = 16
KH = 64
PW = HID * KH
MSGW = 128

CHUNK = 128
NW = 32
NCHUNKS = N_EDGES // CHUNK
CPW = (NCHUNKS + NW - 1) // NW

BE = 1000
BN = 1000


@functools.lru_cache(maxsize=None)
def _sc_kernels():
    mesh = plsc.VectorSubcoreMesh(core_axis_name="c", subcore_axis_name="s")

    @functools.partial(
        pl.kernel,
        out_type=jax.ShapeDtypeStruct((N_EDGES, IN_CH), jnp.float32),
        mesh=mesh,
        scratch_types=[
            pltpu.VMEM((CHUNK,), jnp.int32),
            pltpu.VMEM((CHUNK, IN_CH), jnp.float32),
            pltpu.SemaphoreType.DMA,
        ],
    )
    def _sc_gather(x_hbm, src_hbm, out_hbm, idx_v, rows_v, sem):
        wid = lax.axis_index("s") * 2 + lax.axis_index("c")

        def body(i, carry):
            j = wid + i * NW

            @pl.when(j < NCHUNKS)
            def _():
                base = j * CHUNK
                pltpu.sync_copy(src_hbm.at[pl.ds(base, CHUNK)], idx_v)
                pltpu.async_copy(x_hbm.at[idx_v], rows_v, sem).wait()
                pltpu.sync_copy(rows_v, out_hbm.at[pl.ds(base, CHUNK)])

            return carry

        lax.fori_loop(0, CPW, body, 0)

    @functools.partial(
        pl.kernel,
        out_type=jax.ShapeDtypeStruct((2, N_NODES, MSGW), jnp.float32),
        mesh=mesh,
        scratch_types=[
            pltpu.VMEM((CHUNK,), jnp.int32),
            pltpu.VMEM((CHUNK, MSGW), jnp.float32),
            pltpu.VMEM_SHARED((N_NODES, MSGW), jnp.float32),
        ],
    )
    def _sc_scatter(msg_hbm, dst_hbm, zeros_hbm, out_hbm, idx_v, buf_v, acc_sh):
        cid = lax.axis_index("c")
        sid = lax.axis_index("s")
        wid = sid * 2 + cid

        @pl.when(sid == 0)
        def _():
            pltpu.sync_copy(zeros_hbm, acc_sh)

        plsc.subcore_barrier()

        def body(i, carry):
            j = wid + i * NW

            @pl.when(j < NCHUNKS)
            def _():
                base = j * CHUNK
                pltpu.sync_copy(dst_hbm.at[pl.ds(base, CHUNK)], idx_v)
                pltpu.sync_copy(msg_hbm.at[pl.ds(base, CHUNK)], buf_v)
                pltpu.sync_copy(buf_v, acc_sh.at[idx_v], add=True)

            return carry

        lax.fori_loop(0, CPW, body, 0)
        plsc.subcore_barrier()

        @pl.when(sid == 0)
        def _():
            pltpu.sync_copy(acc_sh, out_hbm.at[cid])

    return _sc_gather, _sc_scatter


def _edge_body(ea_ref, xj_ref, w1_ref, b1_ref, t_ref, s_ref, out_ref):
    h = jnp.dot(ea_ref[...], w1_ref[...], preferred_element_type=jnp.float32)
    h = jnp.maximum(h + b1_ref[...], 0.0)
    p = jnp.dot(xj_ref[...].astype(jnp.bfloat16), t_ref[...],
                preferred_element_type=jnp.float32)
    htile = jnp.concatenate([h] * HID, axis=1)
    g = p[:, :PW] * htile
    msg = jnp.dot(g, s_ref[...], preferred_element_type=jnp.float32)
    msg = msg + p[:, PW:]
    onec = (lax.broadcasted_iota(jnp.int32, (BE, HID), 1) == 0)
    pad = jnp.zeros((BE, MSGW - 2 * HID), dtype=jnp.float32)
    out_ref[...] = jnp.concatenate([msg, onec.astype(jnp.float32), pad], axis=1)


def _edge_kernel(edge_attr, xj, w1, b1_2d, t2, s):
    return pl.pallas_call(
        _edge_body,
        grid=(N_EDGES // BE,),
        in_specs=[
            pl.BlockSpec((BE, HID), lambda i: (i, 0)),
            pl.BlockSpec((BE, IN_CH), lambda i: (i, 0)),
            pl.BlockSpec((HID, KH), lambda i: (0, 0)),
            pl.BlockSpec((1, KH), lambda i: (0, 0)),
            pl.BlockSpec((IN_CH, PW + HID), lambda i: (0, 0)),
            pl.BlockSpec((PW, HID), lambda i: (0, 0)),
        ],
        out_specs=pl.BlockSpec((BE, MSGW), lambda i: (i, 0)),
        out_shape=jax.ShapeDtypeStruct((N_EDGES, MSGW), jnp.float32),
    )(edge_attr, xj, w1, b1_2d, t2, s)


def _final_body(p_ref, x_ref, root_ref, bias_ref, wfc_ref, bfc_ref, out_ref):
    s = p_ref[0] + p_ref[1]
    sums = s[:, :HID]
    cnt = s[:, HID:HID + 1]
    aggr = sums / jnp.maximum(cnt, 1.0)
    conv = aggr + jnp.dot(x_ref[...], root_ref[...],
                          preferred_element_type=jnp.float32) + bias_ref[...]
    conv = jnp.maximum(conv, 0.0)
    out_ref[...] = jnp.dot(conv, wfc_ref[...],
                           preferred_element_type=jnp.float32) + bfc_ref[...]


def _final_kernel(partials, x, root, bias_2d, wfc, bfc_2d):
    return pl.pallas_call(
        _final_body,
        grid=(N_NODES // BN,),
        in_specs=[
            pl.BlockSpec((2, BN, MSGW), lambda i: (0, i, 0)),
            pl.BlockSpec((BN, IN_CH), lambda i: (i, 0)),
            pl.BlockSpec((IN_CH, HID), lambda i: (0, 0)),
            pl.BlockSpec((1, HID), lambda i: (0, 0)),
            pl.BlockSpec((HID, 1), lambda i: (0, 0)),
            pl.BlockSpec((1, 1), lambda i: (0, 0)),
        ],
        out_specs=pl.BlockSpec((BN, 1), lambda i: (i, 0)),
        out_shape=jax.ShapeDtypeStruct((N_NODES, 1), jnp.float32),
    )(partials, x, root, bias_2d, wfc, bfc_2d)


def kernel(x, edge_index, edge_attr, W1, b1, W2, b2, root, bias, Wfc, bfc):
    src = edge_index[0].astype(jnp.int32)
    dst = edge_index[1].astype(jnp.int32)

    t = jnp.transpose(W2.reshape(KH, IN_CH, HID), (1, 2, 0)).reshape(IN_CH, PW)
    t2 = jnp.concatenate([t, b2.reshape(IN_CH, HID)], axis=1).astype(jnp.bfloat16)
    s = jnp.repeat(jnp.eye(HID, dtype=jnp.float32), KH, axis=0)

    sc_gather, sc_scatter = _sc_kernels()
    xj = sc_gather(x, src)
    msg32 = _edge_kernel(edge_attr, xj, W1, b1.reshape(1, KH), t2, s)
    zeros = jnp.zeros((N_NODES, MSGW), dtype=jnp.float32)
    partials = sc_scatter(msg32, dst, zeros)
    return _final_kernel(partials, x, root, bias.reshape(1, HID),
                         Wfc, bfc.reshape(1, 1))


# speedup vs baseline: 1.0153x; 1.0153x over previous
import functools

import jax
import jax.numpy as jnp
from jax import lax
from jax.experimental import pallas as pl
from jax.experimental.pallas import tpu as pltpu
from jax.experimental.pallas import tpu_sc as plsc

N_NODES = 10000
N_EDGES = 160000
IN_CH = 128
HID = 16
KH = 64
PW = HID * KH
MSGW = 128

CHUNK = 128
NW = 32
NCHUNKS = N_EDGES // CHUNK
CPW = (NCHUNKS + NW - 1) // NW

BE = 1000
BN = 1000


@functools.lru_cache(maxsize=None)
def _sc_kernels():
    mesh = plsc.VectorSubcoreMesh(core_axis_name="c", subcore_axis_name="s")

    @functools.partial(
        pl.kernel,
        out_type=jax.ShapeDtypeStruct((N_EDGES, IN_CH), jnp.float32),
        mesh=mesh,
        scratch_types=[
            pltpu.VMEM((CHUNK,), jnp.int32),
            pltpu.VMEM((CHUNK, IN_CH), jnp.float32),
            pltpu.SemaphoreType.DMA,
        ],
    )
    def _sc_gather(x_hbm, src_hbm, out_hbm, idx_v, rows_v, sem):
        wid = lax.axis_index("s") * 2 + lax.axis_index("c")

        def body(i, carry):
            j = wid + i * NW

            @pl.when(j < NCHUNKS)
            def _():
                base = j * CHUNK
                pltpu.sync_copy(src_hbm.at[pl.ds(base, CHUNK)], idx_v)
                pltpu.async_copy(x_hbm.at[idx_v], rows_v, sem).wait()
                pltpu.sync_copy(rows_v, out_hbm.at[pl.ds(base, CHUNK)])

            return carry

        lax.fori_loop(0, CPW, body, 0)

    @functools.partial(
        pl.kernel,
        out_type=jax.ShapeDtypeStruct((2, N_NODES, MSGW), jnp.float32),
        mesh=mesh,
        scratch_types=[
            pltpu.VMEM((CHUNK,), jnp.int32),
            pltpu.VMEM((CHUNK, MSGW), jnp.float32),
            pltpu.VMEM_SHARED((N_NODES, MSGW), jnp.float32),
        ],
    )
    def _sc_scatter(msg_hbm, dst_hbm, zeros_hbm, out_hbm, idx_v, buf_v, acc_sh):
        cid = lax.axis_index("c")
        sid = lax.axis_index("s")
        wid = sid * 2 + cid

        @pl.when(sid == 0)
        def _():
            pltpu.sync_copy(zeros_hbm, acc_sh)

        plsc.subcore_barrier()

        def body(i, carry):
            j = wid + i * NW

            @pl.when(j < NCHUNKS)
            def _():
                base = j * CHUNK
                pltpu.sync_copy(dst_hbm.at[pl.ds(base, CHUNK)], idx_v)
                pltpu.sync_copy(msg_hbm.at[pl.ds(base, CHUNK)], buf_v)
                pltpu.sync_copy(buf_v, acc_sh.at[idx_v], add=True)

            return carry

        lax.fori_loop(0, CPW, body, 0)
        plsc.subcore_barrier()

        @pl.when(sid == 0)
        def _():
            pltpu.sync_copy(acc_sh, out_hbm.at[cid])

    return _sc_gather, _sc_scatter


def _edge_body(ea_ref, xj_ref, w1_ref, b1_ref, t_ref, s_ref, out_ref):
    h = jnp.dot(ea_ref[...], w1_ref[...], preferred_element_type=jnp.float32)
    h = jnp.maximum(h + b1_ref[...], 0.0)
    p = jnp.dot(xj_ref[...], t_ref[...], preferred_element_type=jnp.float32)
    htile = jnp.concatenate([h] * HID, axis=1)
    g = p[:, :PW] * htile
    msg = jnp.dot(g, s_ref[...], preferred_element_type=jnp.float32)
    msg = msg + p[:, PW:]
    onec = (lax.broadcasted_iota(jnp.int32, (BE, HID), 1) == 0)
    pad = jnp.zeros((BE, MSGW - 2 * HID), dtype=jnp.float32)
    out_ref[...] = jnp.concatenate([msg, onec.astype(jnp.float32), pad], axis=1)


def _edge_kernel(edge_attr, xj, w1, b1_2d, t2, s):
    return pl.pallas_call(
        _edge_body,
        grid=(N_EDGES // BE,),
        in_specs=[
            pl.BlockSpec((BE, HID), lambda i: (i, 0)),
            pl.BlockSpec((BE, IN_CH), lambda i: (i, 0)),
            pl.BlockSpec((HID, KH), lambda i: (0, 0)),
            pl.BlockSpec((1, KH), lambda i: (0, 0)),
            pl.BlockSpec((IN_CH, PW + HID), lambda i: (0, 0)),
            pl.BlockSpec((PW, HID), lambda i: (0, 0)),
        ],
        out_specs=pl.BlockSpec((BE, MSGW), lambda i: (i, 0)),
        out_shape=jax.ShapeDtypeStruct((N_EDGES, MSGW), jnp.float32),
    )(edge_attr, xj, w1, b1_2d, t2, s)


def _final_body(p_ref, x_ref, root_ref, bias_ref, wfc_ref, bfc_ref, out_ref):
    s = p_ref[0] + p_ref[1]
    sums = s[:, :HID]
    cnt = s[:, HID:HID + 1]
    aggr = sums / jnp.maximum(cnt, 1.0)
    conv = aggr + jnp.dot(x_ref[...], root_ref[...],
                          preferred_element_type=jnp.float32) + bias_ref[...]
    conv = jnp.maximum(conv, 0.0)
    out_ref[...] = jnp.dot(conv, wfc_ref[...],
                           preferred_element_type=jnp.float32) + bfc_ref[...]


def _final_kernel(partials, x, root, bias_2d, wfc, bfc_2d):
    return pl.pallas_call(
        _final_body,
        grid=(N_NODES // BN,),
        in_specs=[
            pl.BlockSpec((2, BN, MSGW), lambda i: (0, i, 0)),
            pl.BlockSpec((BN, IN_CH), lambda i: (i, 0)),
            pl.BlockSpec((IN_CH, HID), lambda i: (0, 0)),
            pl.BlockSpec((1, HID), lambda i: (0, 0)),
            pl.BlockSpec((HID, 1), lambda i: (0, 0)),
            pl.BlockSpec((1, 1), lambda i: (0, 0)),
        ],
        out_specs=pl.BlockSpec((BN, 1), lambda i: (i, 0)),
        out_shape=jax.ShapeDtypeStruct((N_NODES, 1), jnp.float32),
    )(partials, x, root, bias_2d, wfc, bfc_2d)


def kernel(x, edge_index, edge_attr, W1, b1, W2, b2, root, bias, Wfc, bfc):
    src = edge_index[0].astype(jnp.int32)
    dst = edge_index[1].astype(jnp.int32)

    t = jnp.transpose(W2.reshape(KH, IN_CH, HID), (1, 2, 0)).reshape(IN_CH, PW)
    t2 = jnp.concatenate([t, b2.reshape(IN_CH, HID)], axis=1)
    s = jnp.repeat(jnp.eye(HID, dtype=jnp.float32), KH, axis=0)

    sc_gather, sc_scatter = _sc_kernels()
    xj = sc_gather(x, src)
    msg32 = _edge_kernel(edge_attr, xj, W1, b1.reshape(1, KH), t2, s)
    zeros = jnp.zeros((N_NODES, MSGW), dtype=jnp.float32)
    partials = sc_scatter(msg32, dst, zeros)
    return _final_kernel(partials, x, root, bias.reshape(1, HID),
                         Wfc, bfc.reshape(1, 1))
